# Initial kernel scaffold; baseline (speedup 1.0000x reference)
import jax
import jax.numpy as jnp
from jax.experimental import pallas as pl


def kernel(J, b, W1, b1, W2, b2, W3, b3, W_ih, b_ih, W_hh, b_hh, R1, rb1, R2, rb2, R3, rb3):
    raise NotImplementedError("write your pallas kernel here")


# speedup vs baseline: 38.8128x; 38.8128x over previous
import jax
import jax.numpy as jnp
from jax.experimental import pallas as pl
from jax.experimental.pallas import tpu as pltpu

STATE = 64
HM = 128
NSTEPS = 3
TI = 128
TJ = 128


def _ggnn_body(JT_ref, b_ref,
               W1T_ref, b1_ref,
               W2T_ref, b2_ref, W3T_ref, b3_ref,
               WihrT_ref, WihzT_ref, WihnT_ref, bir_ref, biz_ref, bin_ref,
               WhhrT_ref, WhhzT_ref, WhhnT_ref, bhr_ref, bhz_ref, bhn_ref,
               R1T_ref, rb1_ref, R2T_ref, rb2_ref, R3T_ref, rb3_ref,
               sm_out, ro_out,
               h_s, nm_s):
    s = pl.program_id(0)
    i = pl.program_id(1)
    j = pl.program_id(2)
    NJ = pl.num_programs(2)

    f32 = jnp.float32
    par = jax.lax.rem(s, 2)

    @pl.when((s == 0) & (i == 0) & (j == 0))
    def _init():
        h_s[...] = jnp.zeros_like(h_s)

    @pl.when(j == 0)
    def _init_nm():
        nm_s[...] = jnp.zeros_like(nm_s)

    JTt = JT_ref[...]
    hi_t = h_s[par, pl.ds(i * TI, TI), :]
    hj_t = h_s[par, pl.ds(j * TJ, TJ), :]
    bi_t = b_ref[pl.ds(i * TI, TI), :]
    bj_t = b_ref[pl.ds(j * TJ, TJ), :]
    em = jnp.concatenate([
        jnp.broadcast_to(hi_t[None, :, :], (TJ, TI, STATE)),
        jnp.broadcast_to(hj_t[:, None, :], (TJ, TI, STATE)),
        JTt[:, :, None],
        jnp.broadcast_to(bi_t[None, :, :], (TJ, TI, 1)),
        jnp.broadcast_to(bj_t[:, None, :], (TJ, TI, 1)),
    ], axis=2).reshape(TJ * TI, 2 * STATE + 3)
    x = jnp.maximum(
        jnp.dot(em, W1T_ref[...], preferred_element_type=f32) + b1_ref[...], 0.0)
    x = jnp.maximum(
        jnp.dot(x, W2T_ref[...], preferred_element_type=f32) + b2_ref[...], 0.0)
    m = jnp.dot(x, W3T_ref[...], preferred_element_type=f32) + b3_ref[...]
    mask3 = (JTt != 0.0).astype(f32)[:, :, None]
    m3 = m.reshape(TJ, TI, STATE) * mask3

    acc = nm_s[...]
    for jj in range(TJ):
        acc = acc + m3[jj]
    nm_s[...] = acc

    @pl.when(j == NJ - 1)
    def _gru():
        nm = nm_s[...]
        hi = h_s[par, pl.ds(i * TI, TI), :]
        i_r = jnp.dot(nm, WihrT_ref[...], preferred_element_type=f32) + bir_ref[...]
        i_z = jnp.dot(nm, WihzT_ref[...], preferred_element_type=f32) + biz_ref[...]
        i_n = jnp.dot(nm, WihnT_ref[...], preferred_element_type=f32) + bin_ref[...]
        h_r = jnp.dot(hi, WhhrT_ref[...], preferred_element_type=f32) + bhr_ref[...]
        h_z = jnp.dot(hi, WhhzT_ref[...], preferred_element_type=f32) + bhz_ref[...]
        h_n = jnp.dot(hi, WhhnT_ref[...], preferred_element_type=f32) + bhn_ref[...]
        r = jax.nn.sigmoid(i_r + h_r)
        z = jax.nn.sigmoid(i_z + h_z)
        ng = jnp.tanh(i_n + r * h_n)
        hnew = (1.0 - z) * ng + z * hi
        h_s[1 - par, pl.ds(i * TI, TI), :] = hnew

        @pl.when(s == NSTEPS - 1)
        def _readout():
            ro = jnp.maximum(
                jnp.dot(hnew, R1T_ref[...], preferred_element_type=f32) + rb1_ref[...], 0.0)
            ro = jnp.maximum(
                jnp.dot(ro, R2T_ref[...], preferred_element_type=f32) + rb2_ref[...], 0.0)
            ro = jnp.dot(ro, R3T_ref[...], preferred_element_type=f32) + rb3_ref[...]
            ro_out[pl.ds(i * TI, TI), :] = ro
            mx = jnp.max(ro, axis=1, keepdims=True)
            e = jnp.exp(ro - mx)
            sm_out[pl.ds(i * TI, TI), :] = e / jnp.sum(e, axis=1, keepdims=True)


@jax.jit
def kernel(J, b, W1, b1, W2, b2, W3, b3, W_ih, b_ih, W_hh, b_hh,
           R1, rb1, R2, rb2, R3, rb3):
    N = J.shape[0]
    NI = N // TI
    NJ = N // TJ
    S = STATE

    args = [
        J.T, b[:, None],
        W1.T, b1[None, :],
        W2.T, b2[None, :], W3.T, b3[None, :],
        W_ih[:S].T, W_ih[S:2 * S].T, W_ih[2 * S:].T,
        b_ih[None, :S], b_ih[None, S:2 * S], b_ih[None, 2 * S:],
        W_hh[:S].T, W_hh[S:2 * S].T, W_hh[2 * S:].T,
        b_hh[None, :S], b_hh[None, S:2 * S], b_hh[None, 2 * S:],
        R1.T, rb1[None, :], R2.T, rb2[None, :], R3.T, rb3[None, :],
    ]

    def full(a):
        return pl.BlockSpec(a.shape, lambda s_, i_, j_: (0, 0))

    in_specs = [pl.BlockSpec((TJ, TI), lambda s_, i_, j_: (j_, i_))]
    in_specs += [full(a) for a in args[1:]]

    sm, ro = pl.pallas_call(
        _ggnn_body,
        grid=(NSTEPS, NI, NJ),
        in_specs=in_specs,
        out_specs=[
            pl.BlockSpec((N, 2), lambda s_, i_, j_: (0, 0)),
            pl.BlockSpec((N, 2), lambda s_, i_, j_: (0, 0)),
        ],
        out_shape=[
            jax.ShapeDtypeStruct((N, 2), jnp.float32),
            jax.ShapeDtypeStruct((N, 2), jnp.float32),
        ],
        scratch_shapes=[
            pltpu.VMEM((2, N, S), jnp.float32),
            pltpu.VMEM((TI, S), jnp.float32),
        ],
        compiler_params=pltpu.CompilerParams(
            dimension_semantics=("arbitrary", "arbitrary", "arbitrary"),
        ),
    )(*args)
    return sm, ro
